# initial kernel scaffold (unmeasured)
import jax
import jax.numpy as jnp
from jax import lax
from jax.experimental import pallas as pl
from jax.experimental.pallas import tpu as pltpu

N = 4096
K = 2048
M = 2048
NC = 256
NSTEPS = N // NC


def _body(th_hbm, mi_hbm, wo_hbm, out_hbm,
          th_v, mi_v, wo_buf, send_buf, recv_buf, res_buf,
          in_sems, wo_sems, send_sems, recv_sems, out_sems):
    my_x = lax.axis_index("x")
    my_y = lax.axis_index("y")
    partner = (1 - my_x, my_y)

    cp_th = pltpu.make_async_copy(th_hbm, th_v, in_sems.at[0])
    cp_mi = pltpu.make_async_copy(mi_hbm, mi_v, in_sems.at[1])
    cp_th.start()
    cp_mi.start()
    wo_cp = {}
    wo_cp[0] = pltpu.make_async_copy(
        wo_hbm.at[:, pl.ds(0, NC)], wo_buf.at[0], wo_sems.at[0])
    wo_cp[0].start()

    barrier = pltpu.get_barrier_semaphore()
    pl.semaphore_signal(barrier, inc=1, device_id=partner,
                        device_id_type=pl.DeviceIdType.MESH)
    pl.semaphore_wait(barrier, 1)

    cp_th.wait()
    cp_mi.wait()

    rdmas = {}
    out_cps = {}
    for j in range(NSTEPS):
        slot = j % 2
        wo_cp[j].wait()
        if j + 1 < NSTEPS:
            wo_cp[j + 1] = pltpu.make_async_copy(
                wo_hbm.at[:, pl.ds((j + 1) * NC, NC)],
                wo_buf.at[(j + 1) % 2], wo_sems.at[(j + 1) % 2])
            wo_cp[j + 1].start()

        if j >= 2:
            rdmas[j - 2].wait_send()
        send_buf[slot] = jnp.dot(th_v[...], wo_buf[slot],
                                 preferred_element_type=jnp.float32)
        rdma = pltpu.make_async_remote_copy(
            src_ref=send_buf.at[slot],
            dst_ref=recv_buf.at[slot],
            send_sem=send_sems.at[slot],
            recv_sem=recv_sems.at[slot],
            device_id=partner,
            device_id_type=pl.DeviceIdType.MESH,
        )
        rdma.start()
        rdmas[j] = rdma

        mine_p = jnp.dot(mi_v[...], wo_buf[slot],
                         preferred_element_type=jnp.float32)

        rdma.wait_recv()
        if j >= 2:
            out_cps[j - 2].wait()
        res_buf[slot] = mine_p + recv_buf[slot]
        out_cps[j] = pltpu.make_async_copy(
            res_buf.at[slot], out_hbm.at[:, pl.ds(j * NC, NC)],
            out_sems.at[slot])
        out_cps[j].start()

    rdmas[NSTEPS - 2].wait_send()
    rdmas[NSTEPS - 1].wait_send()
    out_cps[NSTEPS - 2].wait()
    out_cps[NSTEPS - 1].wait()


def kernel(O, Wo):
    B, S2, H, D = O.shape
    S = S2 // 2
    X = O.reshape(B, S2, H * D)
    my_x = lax.axis_index("x")
    mine = lax.dynamic_slice_in_dim(X, my_x * S, S, axis=1)
    theirs = lax.dynamic_slice_in_dim(X, (1 - my_x) * S, S, axis=1)
    mine2d = mine.reshape(B * S, H * D)
    theirs2d = theirs.reshape(B * S, H * D)

    out2d = pl.pallas_call(
        _body,
        out_shape=jax.ShapeDtypeStruct((M, N), jnp.float32),
        in_specs=[
            pl.BlockSpec(memory_space=pltpu.ANY),
            pl.BlockSpec(memory_space=pltpu.ANY),
            pl.BlockSpec(memory_space=pltpu.ANY),
        ],
        out_specs=pl.BlockSpec(memory_space=pltpu.ANY),
        scratch_shapes=[
            pltpu.VMEM((M, K), jnp.float32),
            pltpu.VMEM((M, K), jnp.float32),
            pltpu.VMEM((2, K, NC), jnp.float32),
            pltpu.VMEM((2, M, NC), jnp.float32),
            pltpu.VMEM((2, M, NC), jnp.float32),
            pltpu.VMEM((2, M, NC), jnp.float32),
            pltpu.SemaphoreType.DMA((2,)),
            pltpu.SemaphoreType.DMA((2,)),
            pltpu.SemaphoreType.DMA((2,)),
            pltpu.SemaphoreType.DMA((2,)),
            pltpu.SemaphoreType.DMA((2,)),
        ],
        compiler_params=pltpu.CompilerParams(collective_id=0),
    )(theirs2d, mine2d, Wo)

    return out2d.reshape(B, S, N)


# baseline (device time: 527120 ns/iter reference)
import jax
import jax.numpy as jnp
from jax import lax
from jax.experimental import pallas as pl
from jax.experimental.pallas import tpu as pltpu

N = 4096
K = 2048
M = 2048
NC = 256
NSTEPS = N // NC
RB = 512


def _body(th_hbm, mi_hbm, wo_hbm, out_hbm,
          th_v, mi_v, wo_buf, send_buf, recv_buf, res_buf,
          in_sems, wo_sems, send_sems, recv_sems, out_sems):
    my_x = lax.axis_index("x")
    my_y = lax.axis_index("y")
    partner = (1 - my_x, my_y)

    cp_th = pltpu.make_async_copy(th_hbm, th_v, in_sems.at[0])
    cp_mi = pltpu.make_async_copy(mi_hbm, mi_v, in_sems.at[1])
    cp_th.start()
    cp_mi.start()
    wo_cp = {}
    wo_cp[0] = pltpu.make_async_copy(
        wo_hbm.at[:, pl.ds(0, NC)], wo_buf.at[0], wo_sems.at[0])
    wo_cp[0].start()

    barrier = pltpu.get_barrier_semaphore()
    pl.semaphore_signal(barrier, inc=1, device_id=partner,
                        device_id_type=pl.DeviceIdType.MESH)
    pl.semaphore_wait(barrier, 1)

    cp_th.wait()
    cp_mi.wait()

    rdmas = {}
    out_cps = {}
    for j in range(NSTEPS):
        slot = j % 2
        wo_cp[j].wait()
        if j + 1 < NSTEPS:
            wo_cp[j + 1] = pltpu.make_async_copy(
                wo_hbm.at[:, pl.ds((j + 1) * NC, NC)],
                wo_buf.at[(j + 1) % 2], wo_sems.at[(j + 1) % 2])
            wo_cp[j + 1].start()

        if j >= 2:
            rdmas[j - 2].wait_send()
        for r in range(0, M, RB):
            send_buf[slot, r:r + RB, :] = jnp.dot(
                th_v[r:r + RB, :], wo_buf[slot],
                preferred_element_type=jnp.float32)
        rdma = pltpu.make_async_remote_copy(
            src_ref=send_buf.at[slot],
            dst_ref=recv_buf.at[slot],
            send_sem=send_sems.at[slot],
            recv_sem=recv_sems.at[slot],
            device_id=partner,
            device_id_type=pl.DeviceIdType.MESH,
        )
        rdma.start()
        rdmas[j] = rdma

        if j >= 2:
            out_cps[j - 2].wait()
        for r in range(0, M, RB):
            res_buf[slot, r:r + RB, :] = jnp.dot(
                mi_v[r:r + RB, :], wo_buf[slot],
                preferred_element_type=jnp.float32)

        rdma.wait_recv()
        res_buf[slot] += recv_buf[slot]
        out_cps[j] = pltpu.make_async_copy(
            res_buf.at[slot], out_hbm.at[:, pl.ds(j * NC, NC)],
            out_sems.at[slot])
        out_cps[j].start()

    rdmas[NSTEPS - 2].wait_send()
    rdmas[NSTEPS - 1].wait_send()
    out_cps[NSTEPS - 2].wait()
    out_cps[NSTEPS - 1].wait()


def kernel(O, Wo):
    B, S2, H, D = O.shape
    S = S2 // 2
    X = O.reshape(B, S2, H * D)
    my_x = lax.axis_index("x")
    mine = lax.dynamic_slice_in_dim(X, my_x * S, S, axis=1)
    theirs = lax.dynamic_slice_in_dim(X, (1 - my_x) * S, S, axis=1)
    mine2d = mine.reshape(B * S, H * D)
    theirs2d = theirs.reshape(B * S, H * D)

    out2d = pl.pallas_call(
        _body,
        out_shape=jax.ShapeDtypeStruct((M, N), jnp.float32),
        in_specs=[
            pl.BlockSpec(memory_space=pl.ANY),
            pl.BlockSpec(memory_space=pl.ANY),
            pl.BlockSpec(memory_space=pl.ANY),
        ],
        out_specs=pl.BlockSpec(memory_space=pl.ANY),
        scratch_shapes=[
            pltpu.VMEM((M, K), jnp.float32),
            pltpu.VMEM((M, K), jnp.float32),
            pltpu.VMEM((2, K, NC), jnp.float32),
            pltpu.VMEM((2, M, NC), jnp.float32),
            pltpu.VMEM((2, M, NC), jnp.float32),
            pltpu.VMEM((2, M, NC), jnp.float32),
            pltpu.SemaphoreType.DMA((2,)),
            pltpu.SemaphoreType.DMA((2,)),
            pltpu.SemaphoreType.DMA((2,)),
            pltpu.SemaphoreType.DMA((2,)),
            pltpu.SemaphoreType.DMA((2,)),
        ],
        compiler_params=pltpu.CompilerParams(
            collective_id=0,
            vmem_limit_bytes=60 * 1024 * 1024,
        ),
    )(theirs2d, mine2d, Wo)

    return out2d.reshape(B, S, N)


# device time: 460520 ns/iter; 1.1446x vs baseline; 1.1446x over previous
import jax
import jax.numpy as jnp
from jax import lax
from jax.experimental import pallas as pl
from jax.experimental.pallas import tpu as pltpu

N = 4096
K = 2048
M = 2048
NC = 256
NSTEPS = N // NC
RB = 512


def _body(th_hbm, mi_hbm, wo_hbm, out_hbm,
          th_v, mi_v, wo_buf, send_buf, recv_buf, res_buf,
          in_sems, wo_sems, send_sems, recv_sems, out_sems):
    my_x = lax.axis_index("x")
    my_y = lax.axis_index("y")
    partner = (1 - my_x, my_y)

    cp_th = pltpu.make_async_copy(th_hbm, th_v, in_sems.at[0])
    cp_mi = pltpu.make_async_copy(mi_hbm, mi_v, in_sems.at[1])
    cp_th.start()
    cp_mi.start()
    wo_cp = {}
    wo_cp[0] = pltpu.make_async_copy(
        wo_hbm.at[:, pl.ds(0, NC)], wo_buf.at[0], wo_sems.at[0])
    wo_cp[0].start()

    barrier = pltpu.get_barrier_semaphore()
    pl.semaphore_signal(barrier, inc=1, device_id=partner,
                        device_id_type=pl.DeviceIdType.MESH)
    pl.semaphore_wait(barrier, 1)

    cp_th.wait()
    cp_mi.wait()

    rdmas = {}
    out_cps = {}

    def finish_chunk(j):
        rdmas[j].wait_recv()
        res_buf[j % 2] += recv_buf[j % 4]
        out_cps[j] = pltpu.make_async_copy(
            res_buf.at[j % 2], out_hbm.at[:, pl.ds(j * NC, NC)],
            out_sems.at[j % 2])
        out_cps[j].start()

    for j in range(NSTEPS):
        wo_cp[j].wait()
        if j + 1 < NSTEPS:
            wo_cp[j + 1] = pltpu.make_async_copy(
                wo_hbm.at[:, pl.ds((j + 1) * NC, NC)],
                wo_buf.at[(j + 1) % 2], wo_sems.at[(j + 1) % 2])
            wo_cp[j + 1].start()

        if j >= 2:
            rdmas[j - 2].wait_send()
        for r in range(0, M, RB):
            send_buf[j % 2, r:r + RB, :] = jnp.dot(
                th_v[r:r + RB, :], wo_buf[j % 2],
                preferred_element_type=jnp.float32)
        rdma = pltpu.make_async_remote_copy(
            src_ref=send_buf.at[j % 2],
            dst_ref=recv_buf.at[j % 4],
            send_sem=send_sems.at[j % 2],
            recv_sem=recv_sems.at[j % 4],
            device_id=partner,
            device_id_type=pl.DeviceIdType.MESH,
        )
        rdma.start()
        rdmas[j] = rdma

        if j >= 2:
            out_cps[j - 2].wait()
        for r in range(0, M, RB):
            res_buf[j % 2, r:r + RB, :] = jnp.dot(
                mi_v[r:r + RB, :], wo_buf[j % 2],
                preferred_element_type=jnp.float32)

        if j >= 1:
            finish_chunk(j - 1)

    finish_chunk(NSTEPS - 1)
    rdmas[NSTEPS - 2].wait_send()
    rdmas[NSTEPS - 1].wait_send()
    out_cps[NSTEPS - 2].wait()
    out_cps[NSTEPS - 1].wait()


def kernel(O, Wo):
    B, S2, H, D = O.shape
    S = S2 // 2
    X = O.reshape(B, S2, H * D)
    my_x = lax.axis_index("x")
    mine = lax.dynamic_slice_in_dim(X, my_x * S, S, axis=1)
    theirs = lax.dynamic_slice_in_dim(X, (1 - my_x) * S, S, axis=1)
    mine2d = mine.reshape(B * S, H * D)
    theirs2d = theirs.reshape(B * S, H * D)

    out2d = pl.pallas_call(
        _body,
        out_shape=jax.ShapeDtypeStruct((M, N), jnp.float32),
        in_specs=[
            pl.BlockSpec(memory_space=pl.ANY),
            pl.BlockSpec(memory_space=pl.ANY),
            pl.BlockSpec(memory_space=pl.ANY),
        ],
        out_specs=pl.BlockSpec(memory_space=pl.ANY),
        scratch_shapes=[
            pltpu.VMEM((M, K), jnp.float32),
            pltpu.VMEM((M, K), jnp.float32),
            pltpu.VMEM((2, K, NC), jnp.float32),
            pltpu.VMEM((2, M, NC), jnp.float32),
            pltpu.VMEM((4, M, NC), jnp.float32),
            pltpu.VMEM((2, M, NC), jnp.float32),
            pltpu.SemaphoreType.DMA((2,)),
            pltpu.SemaphoreType.DMA((2,)),
            pltpu.SemaphoreType.DMA((2,)),
            pltpu.SemaphoreType.DMA((4,)),
            pltpu.SemaphoreType.DMA((2,)),
        ],
        compiler_params=pltpu.CompilerParams(
            collective_id=0,
            vmem_limit_bytes=62 * 1024 * 1024,
        ),
    )(theirs2d, mine2d, Wo)

    return out2d.reshape(B, S, N)


# device time: 281235 ns/iter; 1.8743x vs baseline; 1.6375x over previous
import jax
import jax.numpy as jnp
from jax import lax
from jax.experimental import pallas as pl
from jax.experimental.pallas import tpu as pltpu

N = 4096
K = 2048
B = 4
SH = 512
NC = 256
NPAIR = (N // NC) // 2


def _body(x_hbm, wo_hbm, out_hbm,
          th_v, mi_v, wo_buf, send_x, recv_x, res_buf, recv_y,
          st_sems, wo_sems, sx_sems, rx_sems, out_sems, ys_sems,
          ry_sems, oy_sems):
    my_x = lax.axis_index("x")
    my_y = lax.axis_index("y")
    xp = (1 - my_x, my_y)
    yp = (my_x, 1 - my_y)

    def jd_off(k):
        return 2 * k * NC + my_y * NC

    def jo_off(k):
        return 2 * k * NC + (1 - my_y) * NC

    st_cp = []
    for b in range(B):
        for dst, row in ((mi_v, b * 2 * SH + my_x * SH),
                         (th_v, b * 2 * SH + (1 - my_x) * SH)):
            cp = pltpu.make_async_copy(
                x_hbm.at[pl.ds(row, SH), :], dst.at[b],
                st_sems.at[len(st_cp) % 2])
            st_cp.append(cp)

    st_cp[0].start()
    st_cp[1].start()

    wo_cp = {}
    wo_cp[0] = pltpu.make_async_copy(
        wo_hbm.at[:, pl.ds(jd_off(0), NC)], wo_buf.at[0], wo_sems.at[0])
    wo_cp[0].start()

    barrier = pltpu.get_barrier_semaphore()
    for nbr in (xp, yp):
        pl.semaphore_signal(barrier, inc=1, device_id=nbr,
                            device_id_type=pl.DeviceIdType.MESH)
    pl.semaphore_wait(barrier, 2)

    for i in range(len(st_cp)):
        st_cp[i].wait()
        if i + 2 < len(st_cp):
            st_cp[i + 2].start()

    rdx = {}
    out_cps = {}
    ysend = {}
    outy = {}

    def process_direct(p):
        rdx[p].wait_recv()
        res_buf[p % 4] += recv_x[p % 4]
        ysend[p] = pltpu.make_async_remote_copy(
            src_ref=res_buf.at[p % 4],
            dst_ref=recv_y.at[p % 4],
            send_sem=ys_sems.at[p % 4],
            recv_sem=ry_sems.at[p % 4],
            device_id=yp,
            device_id_type=pl.DeviceIdType.MESH,
        )
        ysend[p].start()
        out_cps[p] = pltpu.make_async_copy(
            res_buf.at[p % 4], out_hbm.at[:, :, pl.ds(jd_off(p), NC)],
            out_sems.at[p % 4])
        out_cps[p].start()

    def process_fwd(q):
        r = pltpu.make_async_remote_copy(
            src_ref=res_buf.at[q % 4],
            dst_ref=recv_y.at[q % 4],
            send_sem=ys_sems.at[q % 4],
            recv_sem=ry_sems.at[q % 4],
            device_id=yp,
            device_id_type=pl.DeviceIdType.MESH,
        )
        r.wait_recv()
        if q >= 4:
            outy[q - 4].wait()
        outy[q] = pltpu.make_async_copy(
            recv_y.at[q % 4], out_hbm.at[:, :, pl.ds(jo_off(q), NC)],
            oy_sems.at[q % 4])
        outy[q].start()

    for k in range(NPAIR):
        wo_cp[k].wait()
        if k + 1 < NPAIR:
            wo_cp[k + 1] = pltpu.make_async_copy(
                wo_hbm.at[:, pl.ds(jd_off(k + 1), NC)],
                wo_buf.at[(k + 1) % 2], wo_sems.at[(k + 1) % 2])
            wo_cp[k + 1].start()

        if k >= 2:
            rdx[k - 2].wait_send()
        for b in range(B):
            send_x[k % 2, b] = jnp.dot(
                th_v[b], wo_buf[k % 2], preferred_element_type=jnp.float32)
        rdx[k] = pltpu.make_async_remote_copy(
            src_ref=send_x.at[k % 2],
            dst_ref=recv_x.at[k % 4],
            send_sem=sx_sems.at[k % 2],
            recv_sem=rx_sems.at[k % 4],
            device_id=xp,
            device_id_type=pl.DeviceIdType.MESH,
        )
        rdx[k].start()

        if k >= 4:
            out_cps[k - 4].wait()
            ysend[k - 4].wait_send()
        for b in range(B):
            res_buf[k % 4, b] = jnp.dot(
                mi_v[b], wo_buf[k % 2], preferred_element_type=jnp.float32)

        if k >= 1:
            process_direct(k - 1)
        if k >= 2:
            process_fwd(k - 2)

    process_direct(NPAIR - 1)
    process_fwd(NPAIR - 2)
    process_fwd(NPAIR - 1)
    rdx[NPAIR - 2].wait_send()
    rdx[NPAIR - 1].wait_send()
    for p in range(NPAIR - 4, NPAIR):
        out_cps[p].wait()
        ysend[p].wait_send()
        outy[p].wait()


def kernel(O, Wo):
    Bv, S2, H, D = O.shape
    X = O.reshape(Bv * S2, H * D).astype(jnp.bfloat16)
    Wo = Wo.astype(jnp.bfloat16)

    return pl.pallas_call(
        _body,
        out_shape=jax.ShapeDtypeStruct((B, SH, N), jnp.float32),
        in_specs=[
            pl.BlockSpec(memory_space=pl.ANY),
            pl.BlockSpec(memory_space=pl.ANY),
        ],
        out_specs=pl.BlockSpec(memory_space=pl.ANY),
        scratch_shapes=[
            pltpu.VMEM((B, SH, K), jnp.bfloat16),
            pltpu.VMEM((B, SH, K), jnp.bfloat16),
            pltpu.VMEM((2, K, NC), jnp.bfloat16),
            pltpu.VMEM((2, B, SH, NC), jnp.float32),
            pltpu.VMEM((4, B, SH, NC), jnp.float32),
            pltpu.VMEM((4, B, SH, NC), jnp.float32),
            pltpu.VMEM((4, B, SH, NC), jnp.float32),
            pltpu.SemaphoreType.DMA((2,)),
            pltpu.SemaphoreType.DMA((2,)),
            pltpu.SemaphoreType.DMA((2,)),
            pltpu.SemaphoreType.DMA((4,)),
            pltpu.SemaphoreType.DMA((4,)),
            pltpu.SemaphoreType.DMA((4,)),
            pltpu.SemaphoreType.DMA((4,)),
            pltpu.SemaphoreType.DMA((4,)),
        ],
        compiler_params=pltpu.CompilerParams(
            collective_id=0,
            vmem_limit_bytes=62 * 1024 * 1024,
        ),
    )(X, Wo)
